# baseline (speedup 1.0000x reference)
import functools

import jax
import jax.numpy as jnp
from jax import lax
from jax.experimental import pallas as pl
from jax.experimental.pallas import tpu as pltpu
from jax.experimental.pallas import tpu_sc as plsc

NEG = 0.2
N = 10000
E = 320000
D = 128
NC = 2
NS = 16
NW = NC * NS
EPW = E // NW
BLK = 80
NBLK = EPW // BLK
GRP = BLK // 16
WCH = 1000
WNT = N // WCH
ZCH = 250


def _take(v, idx):
    return v.at[idx].get(mode="promise_in_bounds")


def _edge_body(xl_hbm, xr_hbm, src_hbm, dst_hbm, ea_hbm, vm_hbm,
               outA, outB, accA, accB, srcb, dstb, eab, vmb,
               rows_l, rows_r, valB, wv, attv, sg0, sg1, ss0, ss1, sc0, sc1):
    c = lax.axis_index("c")
    s = lax.axis_index("s")
    wid = c * NS + s
    zero16 = jnp.zeros((16,), jnp.float32)
    ebase = wid * EPW

    def _zrow(i, _):
        for j in range(8):
            rows_l[0, i, pl.ds(16 * j, 16)] = zero16
        return 0
    lax.fori_loop(0, BLK, _zrow, 0)
    for i in range(BLK // 16):
        valB[0, pl.ds(i * 16, 16)] = zero16

    @pl.when(s < WNT)
    def _():
        for t in range(WCH // 40):
            pltpu.sync_copy(rows_l.at[0, pl.ds(0, 40)],
                            accA.at[pl.ds(s * WCH + t * 40, 40)])
            pltpu.sync_copy(valB.at[0, pl.ds(0, 40)],
                            accB.at[pl.ds(s * WCH + t * 40, 40)])
    plsc.subcore_barrier()

    lane = lax.iota(jnp.int32, 16)
    sgs = (sg0, sg1)
    sss = (ss0, ss1)
    scs = (sc0, sc1)

    def issue_scalars(n, q, sem):
        base = ebase + n * BLK
        pltpu.async_copy(src_hbm.at[pl.ds(base, BLK)], srcb.at[q], sem)
        pltpu.async_copy(dst_hbm.at[pl.ds(base, BLK)], dstb.at[q], sem)
        pltpu.async_copy(ea_hbm.at[pl.ds(base, BLK)], eab.at[q], sem)
        pltpu.async_copy(vm_hbm.at[pl.ds(base, BLK)], vmb.at[q], sem)

    def drain_scalars(sem):
        pltpu.make_async_copy(src_hbm.at[pl.ds(0, BLK)], srcb.at[0], sem).wait()
        pltpu.make_async_copy(dst_hbm.at[pl.ds(0, BLK)], dstb.at[0], sem).wait()
        pltpu.make_async_copy(ea_hbm.at[pl.ds(0, BLK)], eab.at[0], sem).wait()
        pltpu.make_async_copy(vm_hbm.at[pl.ds(0, BLK)], vmb.at[0], sem).wait()

    def issue_gathers(q, p, sem):
        pltpu.async_copy(xl_hbm.at[srcb.at[q]], rows_l.at[p], sem)
        pltpu.async_copy(xr_hbm.at[dstb.at[q]], rows_r.at[p], sem)

    def drain_gathers(p, sem):
        pltpu.make_async_copy(xl_hbm.at[pl.ds(0, BLK)], rows_l.at[p], sem).wait()
        pltpu.make_async_copy(xr_hbm.at[pl.ds(0, BLK)], rows_r.at[p], sem).wait()

    def drain_scatter(p):
        pltpu.make_async_copy(rows_l.at[p], accA.at[dstb.at[0]],
                              scs[p]).wait()
        pltpu.make_async_copy(valB.at[p], accB.at[dstb.at[0]],
                              scs[p]).wait()

    def compute_scatter(n, p):
        q = lax.rem(n, 3)
        wregs = [wv[pl.ds(16 * j, 16)] for j in range(8)]
        aregs = [attv[pl.ds(16 * j, 16)] for j in range(8)]

        def _group(g, _):
            eav = eab[q, pl.ds(g * 16, 16)]
            vmv = vmb[q, pl.ds(g * 16, 16)]
            valB[p, pl.ds(g * 16, 16)] = zero16
            for k in range(16):
                e = g * 16 + k
                kk = jnp.full((16,), k, jnp.int32)
                vms = vmv[k]

                @pl.when(vms > 0.0)
                def _(e=e, kk=kk):
                    ea_b = _take(eav, kk)
                    sacc = zero16
                    lch = []
                    for j in range(8):
                        lj = rows_l[p, e, pl.ds(16 * j, 16)]
                        rj = rows_r[p, e, pl.ds(16 * j, 16)]
                        lch.append(lj)
                        z = lj + rj + ea_b * wregs[j]
                        z = jnp.maximum(z, NEG * z)
                        sacc = sacc + z * aregs[j]
                    for sh in (1, 2, 4, 8):
                        sacc = sacc + _take(sacc, lane ^ sh)
                    pb = jnp.exp(sacc)
                    for j in range(8):
                        rows_l[p, e, pl.ds(16 * j, 16)] = pb * lch[j]
                    valB[p, pl.ds(g * 16, 16)] = jnp.where(
                        lane == kk, pb, valB[p, pl.ds(g * 16, 16)])

                @pl.when(vms <= 0.0)
                def _(e=e):
                    for j in range(8):
                        rows_l[p, e, pl.ds(16 * j, 16)] = zero16
            return 0

        lax.fori_loop(0, GRP, _group, 0)
        pltpu.async_copy(rows_l.at[p], accA.at[dstb.at[q]], scs[p], add=True)
        pltpu.async_copy(valB.at[p], accB.at[dstb.at[q]], scs[p], add=True)

    issue_scalars(0, 0, sss[0])
    drain_scalars(sss[0])
    issue_gathers(0, 0, sgs[0])
    issue_scalars(1, 1, sss[1])

    def _pair(i, _):
        for u in (0, 1):
            n = 2 * i + u
            p = u
            drain_gathers(p, sgs[p])
            drain_scalars(sss[1 - p])

            @pl.when(n >= 1)
            def _():
                drain_scatter(1 - p)
            issue_gathers(lax.rem(n + 1, 3), 1 - p, sgs[1 - p])

            @pl.when(n <= NBLK - 3)
            def _():
                issue_scalars(n + 2, lax.rem(n + 2, 3), sss[p])
            compute_scatter(n, p)
        return 0

    lax.fori_loop(0, (NBLK - 1) // 2, _pair, 0)
    drain_gathers(0, sgs[0])
    drain_scatter(1)
    compute_scatter(NBLK - 1, 0)
    drain_scatter(0)
    plsc.subcore_barrier()

    @pl.when(s < WNT)
    def _():
        pltpu.sync_copy(accA.at[pl.ds(s * WCH, WCH)],
                        outA.at[c, pl.ds(s * WCH, WCH)])
        for t in range(WCH // 40):
            pltpu.sync_copy(accB.at[pl.ds(s * WCH + t * 40, 40)],
                            valB.at[0, pl.ds(0, 40)])
            pltpu.sync_copy(valB.at[0, pl.ds(0, 40)],
                            outB.at[pl.ds(c * N + s * WCH + t * 40, 40)])


def _edge_body_pre(xl_hbm, xr_hbm, src_hbm, dst_hbm, ea_hbm, vm_hbm,
                   w_hbm, att_hbm, outA, outB, accA, accB, srcb, dstb, eab,
                   vmb, rows_l, rows_r, valB, wv, attv,
                   sg0, sg1, ss0, ss1, sc0, sc1):
    pltpu.sync_copy(w_hbm, wv)
    pltpu.sync_copy(att_hbm, attv)
    _edge_body(xl_hbm, xr_hbm, src_hbm, dst_hbm, ea_hbm, vm_hbm,
               outA, outB, accA, accB, srcb, dstb, eab, vmb,
               rows_l, rows_r, valB, wv, attv, sg0, sg1, ss0, ss1, sc0, sc1)


@functools.partial(jax.jit, static_argnames=())
def _sc_edge_pass(xl, xr, src, dst, ea, vm, w, att):
    mesh = plsc.VectorSubcoreMesh(core_axis_name="c", subcore_axis_name="s")
    kern = pl.kernel(
        _edge_body_pre,
        out_type=(
            jax.ShapeDtypeStruct((NC, N, D), jnp.float32),
            jax.ShapeDtypeStruct((NC * N,), jnp.float32),
        ),
        mesh=mesh,
        compiler_params=pltpu.CompilerParams(use_tc_tiling_on_sc=False),
        scratch_types=[
            pltpu.VMEM_SHARED((N, D), jnp.float32),
            pltpu.VMEM_SHARED((N,), jnp.float32),
            pltpu.VMEM((3, BLK), jnp.int32),
            pltpu.VMEM((3, BLK), jnp.int32),
            pltpu.VMEM((3, BLK), jnp.float32),
            pltpu.VMEM((3, BLK), jnp.float32),
            pltpu.VMEM((2, BLK, D), jnp.float32),
            pltpu.VMEM((2, BLK, D), jnp.float32),
            pltpu.VMEM((2, BLK), jnp.float32),
            pltpu.VMEM((D,), jnp.float32),
            pltpu.VMEM((D,), jnp.float32),
            pltpu.SemaphoreType.DMA,
            pltpu.SemaphoreType.DMA,
            pltpu.SemaphoreType.DMA,
            pltpu.SemaphoreType.DMA,
            pltpu.SemaphoreType.DMA,
            pltpu.SemaphoreType.DMA,
        ],
    )
    return kern(xl, xr, src, dst, ea, vm, w, att)


_RB = 1000
_GRID = N // _RB


def _prep_math(h, Wl, bl, Wr, br, wv, att):
    xl = lax.dot_general(h, Wl, (((1,), (1,)), ((), ())),
                         preferred_element_type=jnp.float32) + bl
    xr = lax.dot_general(h, Wr, (((1,), (1,)), ((), ())),
                         preferred_element_type=jnp.float32) + br
    z = xl + xr + wv
    z = jnp.maximum(z, NEG * z)
    a = jnp.sum(z * att, axis=1, keepdims=True)
    p = jnp.exp(a)
    return xl, xr, xl * p, jnp.broadcast_to(p, (h.shape[0], 16))


def _tc_prep_body(x_ref, wl_ref, bl_ref, wr_ref, br_ref, w_ref, att_ref,
                  xl_o, xr_o, sA_o, sB_o):
    xl, xr, sA, sB = _prep_math(x_ref[...], wl_ref[...], bl_ref[...],
                                wr_ref[...], br_ref[...], w_ref[...],
                                att_ref[...])
    xl_o[...] = xl
    xr_o[...] = xr
    sA_o[...] = sA
    sB_o[...] = sB


def _wspec():
    return pl.BlockSpec((D, D), lambda i: (0, 0))


def _vspec():
    return pl.BlockSpec((1, D), lambda i: (0, 0))


def _rspec(w=D):
    return pl.BlockSpec((_RB, w), lambda i: (i, 0))


def _aspec(w=D):
    return pl.BlockSpec((NC, _RB, w), lambda i: (0, i, 0))


def _bspec():
    return pl.BlockSpec((NC, _RB, 1), lambda i: (0, i, 0))


def _den_slice(aB_ref, sB_ref):
    aB = aB_ref[...]
    return aB[0] + aB[1] + sB_ref[:, 0:1]


def _tc_prep(x, Wl, bl, Wr, br, w, att):
    return pl.pallas_call(
        _tc_prep_body,
        grid=(_GRID,),
        in_specs=[_rspec(), _wspec(), _vspec(), _wspec(), _vspec(),
                  _vspec(), _vspec()],
        out_specs=[_rspec(), _rspec(), _rspec(), _rspec(16)],
        out_shape=[
            jax.ShapeDtypeStruct((N, D), jnp.float32),
            jax.ShapeDtypeStruct((N, D), jnp.float32),
            jax.ShapeDtypeStruct((N, D), jnp.float32),
            jax.ShapeDtypeStruct((N, 16), jnp.float32),
        ],
    )(x, Wl, bl.reshape(1, D), Wr, br.reshape(1, D), w.reshape(1, D),
      att.reshape(1, D))


def _tc_combine_prep_body(aA_ref, aB_ref, sA_ref, sB_ref, bias_ref,
                          wl_ref, bl_ref, wr_ref, br_ref, w_ref, att_ref,
                          xl_o, xr_o, sA_o, sB_o):
    aA = aA_ref[...]
    num = aA[0] + aA[1] + sA_ref[...]
    den = _den_slice(aB_ref, sB_ref)
    h = num / den + bias_ref[...]
    xl, xr, sA, sB = _prep_math(h, wl_ref[...], bl_ref[...], wr_ref[...],
                                br_ref[...], w_ref[...], att_ref[...])
    xl_o[...] = xl
    xr_o[...] = xr
    sA_o[...] = sA
    sB_o[...] = sB


def _tc_combine_prep(aA, aB, sA, sB, bias, Wl, bl, Wr, br, w, att):
    return pl.pallas_call(
        _tc_combine_prep_body,
        grid=(_GRID,),
        in_specs=[_aspec(), _bspec(), _rspec(), _rspec(16), _vspec(),
                  _wspec(), _vspec(), _wspec(), _vspec(), _vspec(), _vspec()],
        out_specs=[_rspec(), _rspec(), _rspec(), _rspec(16)],
        out_shape=[
            jax.ShapeDtypeStruct((N, D), jnp.float32),
            jax.ShapeDtypeStruct((N, D), jnp.float32),
            jax.ShapeDtypeStruct((N, D), jnp.float32),
            jax.ShapeDtypeStruct((N, 16), jnp.float32),
        ],
    )(aA, aB, sA, sB, bias.reshape(1, D), Wl, bl.reshape(1, D), Wr,
      br.reshape(1, D), w.reshape(1, D), att.reshape(1, D))


def _tc_final_body(aA_ref, aB_ref, sA_ref, sB_ref, bias_ref, out_o):
    aA = aA_ref[...]
    num = aA[0] + aA[1] + sA_ref[...]
    den = _den_slice(aB_ref, sB_ref)
    out_o[...] = num / den + bias_ref[...]


def _tc_final(aA, aB, sA, sB, bias):
    return pl.pallas_call(
        _tc_final_body,
        grid=(_GRID,),
        in_specs=[_aspec(), _bspec(), _rspec(), _rspec(16), _vspec()],
        out_specs=_rspec(),
        out_shape=jax.ShapeDtypeStruct((N, D), jnp.float32),
    )(aA, aB, sA, sB, bias.reshape(1, D))


def kernel(x, edge_index, edge_attr, edge_mask_ii, edge_mask_uiu,
           edge_mask_train, node_mask_item, Wl_ii, bl_ii, Wr_ii, br_ii,
           We_ii, att_ii, bias_ii, Wl_uiu, bl_uiu, Wr_uiu, br_uiu, We_uiu,
           att_uiu, bias_uiu):
    src = edge_index[0]
    dst = edge_index[1]
    ea = edge_attr[:, 0]
    vm1 = edge_mask_ii.astype(jnp.float32)
    vm2 = (edge_mask_uiu & edge_mask_train).astype(jnp.float32)
    w1 = We_ii[:, 0]
    w2 = We_uiu[:, 0]

    xl1, xr1, sA1, sB1 = _tc_prep(x, Wl_ii, bl_ii, Wr_ii, br_ii, w1, att_ii)
    aA1, aB1 = _sc_edge_pass(xl1, xr1, src, dst, ea, vm1, w1, att_ii)
    xl2, xr2, sA2, sB2 = _tc_combine_prep(aA1, aB1.reshape(NC, N, 1), sA1,
                                          sB1, bias_ii, Wl_uiu, bl_uiu,
                                          Wr_uiu, br_uiu, w2, att_uiu)
    aA2, aB2 = _sc_edge_pass(xl2, xr2, src, dst, ea, vm2, w2, att_uiu)
    return _tc_final(aA2, aB2.reshape(NC, N, 1), sA2, sB2, bias_uiu)


# speedup vs baseline: 1.6145x; 1.6145x over previous
import functools

import jax
import jax.numpy as jnp
from jax import lax
from jax.experimental import pallas as pl
from jax.experimental.pallas import tpu as pltpu
from jax.experimental.pallas import tpu_sc as plsc

NEG = 0.2
N = 10000
E = 320000
D = 128
NC = 2
NS = 16
NW = NC * NS
EPW = E // NW
BLK = 80
NBLK = EPW // BLK
GRP = BLK // 16
WCH = 1000
WNT = N // WCH
ZCH = 250


def _take(v, idx):
    return v.at[idx].get(mode="promise_in_bounds")


def _edge_body(xl_hbm, xr_hbm, src_hbm, dst_hbm, ea_hbm, vm_hbm,
               outA, outB, accA, accB, srcb, dstb, eab, vmb,
               rows_l, rows_r, valB, wv, attv, sg0, sg1, ss0, ss1, sc0, sc1):
    c = lax.axis_index("c")
    s = lax.axis_index("s")
    wid = c * NS + s
    zero16 = jnp.zeros((16,), jnp.float32)
    ebase = wid * EPW

    def _zrow(i, _):
        for j in range(8):
            rows_l[0, i, pl.ds(16 * j, 16)] = zero16
        return 0
    lax.fori_loop(0, BLK, _zrow, 0)
    for i in range(BLK // 16):
        valB[0, pl.ds(i * 16, 16)] = zero16

    @pl.when(s < WNT)
    def _():
        for t in range(WCH // 40):
            pltpu.sync_copy(rows_l.at[0, pl.ds(0, 40)],
                            accA.at[pl.ds(s * WCH + t * 40, 40)])
            pltpu.sync_copy(valB.at[0, pl.ds(0, 40)],
                            accB.at[pl.ds(s * WCH + t * 40, 40)])
    plsc.subcore_barrier()

    lane = lax.iota(jnp.int32, 16)
    sgs = (sg0, sg1)
    sss = (ss0, ss1)
    scs = (sc0, sc1)

    def issue_scalars(n, q, sem):
        base = ebase + n * BLK
        pltpu.async_copy(src_hbm.at[pl.ds(base, BLK)], srcb.at[q], sem)
        pltpu.async_copy(dst_hbm.at[pl.ds(base, BLK)], dstb.at[q], sem)
        pltpu.async_copy(ea_hbm.at[pl.ds(base, BLK)], eab.at[q], sem)
        pltpu.async_copy(vm_hbm.at[pl.ds(base, BLK)], vmb.at[q], sem)

    def drain_scalars(sem):
        pltpu.make_async_copy(src_hbm.at[pl.ds(0, BLK)], srcb.at[0], sem).wait()
        pltpu.make_async_copy(dst_hbm.at[pl.ds(0, BLK)], dstb.at[0], sem).wait()
        pltpu.make_async_copy(ea_hbm.at[pl.ds(0, BLK)], eab.at[0], sem).wait()
        pltpu.make_async_copy(vm_hbm.at[pl.ds(0, BLK)], vmb.at[0], sem).wait()

    def issue_gathers(q, p, sem):
        pltpu.async_copy(xl_hbm.at[srcb.at[q]], rows_l.at[p], sem)
        pltpu.async_copy(xr_hbm.at[dstb.at[q]], rows_r.at[p], sem)

    def drain_gathers(p, sem):
        pltpu.make_async_copy(xl_hbm.at[pl.ds(0, BLK)], rows_l.at[p], sem).wait()
        pltpu.make_async_copy(xr_hbm.at[pl.ds(0, BLK)], rows_r.at[p], sem).wait()

    def drain_scatter(p):
        pltpu.make_async_copy(rows_l.at[p], accA.at[dstb.at[0]],
                              scs[p]).wait()
        pltpu.make_async_copy(valB.at[p], accB.at[dstb.at[0]],
                              scs[p]).wait()

    def compute_scatter(n, p):
        q = lax.rem(n, 3)
        wregs = [wv[pl.ds(16 * j, 16)] for j in range(8)]
        aregs = [attv[pl.ds(16 * j, 16)] for j in range(8)]

        def _group(g, _):
            eav = eab[q, pl.ds(g * 16, 16)]
            vmv = vmb[q, pl.ds(g * 16, 16)]
            pvec = zero16
            for k in range(16):
                e = g * 16 + k
                kk = jnp.full((16,), k, jnp.int32)
                ea_b = _take(eav, kk)
                vm_b = _take(vmv, kk)
                sacc = zero16
                lch = []
                for j in range(8):
                    lj = rows_l[p, e, pl.ds(16 * j, 16)]
                    rj = rows_r[p, e, pl.ds(16 * j, 16)]
                    lch.append(lj)
                    z = lj + rj + ea_b * wregs[j]
                    z = jnp.maximum(z, NEG * z)
                    sacc = sacc + z * aregs[j]
                for sh in (1, 2, 4, 8):
                    sacc = sacc + _take(sacc, lane ^ sh)
                pb = jnp.exp(sacc) * vm_b
                for j in range(8):
                    rows_l[p, e, pl.ds(16 * j, 16)] = pb * lch[j]
                pvec = jnp.where(lane == kk, pb, pvec)
            valB[p, pl.ds(g * 16, 16)] = pvec
            return 0

        lax.fori_loop(0, GRP, _group, 0)
        pltpu.async_copy(rows_l.at[p], accA.at[dstb.at[q]], scs[p], add=True)
        pltpu.async_copy(valB.at[p], accB.at[dstb.at[q]], scs[p], add=True)

    issue_scalars(0, 0, sss[0])
    drain_scalars(sss[0])
    issue_gathers(0, 0, sgs[0])
    issue_scalars(1, 1, sss[1])

    def _pair(i, _):
        for u in (0, 1):
            n = 2 * i + u
            p = u
            drain_gathers(p, sgs[p])
            drain_scalars(sss[1 - p])

            @pl.when(n >= 1)
            def _():
                drain_scatter(1 - p)
            issue_gathers(lax.rem(n + 1, 3), 1 - p, sgs[1 - p])

            @pl.when(n <= NBLK - 3)
            def _():
                issue_scalars(n + 2, lax.rem(n + 2, 3), sss[p])
            compute_scatter(n, p)
        return 0

    lax.fori_loop(0, (NBLK - 1) // 2, _pair, 0)
    drain_gathers(0, sgs[0])
    drain_scatter(1)
    compute_scatter(NBLK - 1, 0)
    drain_scatter(0)
    plsc.subcore_barrier()

    @pl.when(s < WNT)
    def _():
        pltpu.sync_copy(accA.at[pl.ds(s * WCH, WCH)],
                        outA.at[c, pl.ds(s * WCH, WCH)])
        for t in range(WCH // 40):
            pltpu.sync_copy(accB.at[pl.ds(s * WCH + t * 40, 40)],
                            valB.at[0, pl.ds(0, 40)])
            pltpu.sync_copy(valB.at[0, pl.ds(0, 40)],
                            outB.at[pl.ds(c * N + s * WCH + t * 40, 40)])


def _edge_body_pre(xl_hbm, xr_hbm, src_hbm, dst_hbm, ea_hbm, vm_hbm,
                   w_hbm, att_hbm, outA, outB, accA, accB, srcb, dstb, eab,
                   vmb, rows_l, rows_r, valB, wv, attv,
                   sg0, sg1, ss0, ss1, sc0, sc1):
    pltpu.sync_copy(w_hbm, wv)
    pltpu.sync_copy(att_hbm, attv)
    _edge_body(xl_hbm, xr_hbm, src_hbm, dst_hbm, ea_hbm, vm_hbm,
               outA, outB, accA, accB, srcb, dstb, eab, vmb,
               rows_l, rows_r, valB, wv, attv, sg0, sg1, ss0, ss1, sc0, sc1)


@functools.partial(jax.jit, static_argnames=())
def _sc_edge_pass(xl, xr, src, dst, ea, vm, w, att):
    mesh = plsc.VectorSubcoreMesh(core_axis_name="c", subcore_axis_name="s")
    kern = pl.kernel(
        _edge_body_pre,
        out_type=(
            jax.ShapeDtypeStruct((NC, N, D), jnp.float32),
            jax.ShapeDtypeStruct((NC * N,), jnp.float32),
        ),
        mesh=mesh,
        compiler_params=pltpu.CompilerParams(use_tc_tiling_on_sc=False),
        scratch_types=[
            pltpu.VMEM_SHARED((N, D), jnp.float32),
            pltpu.VMEM_SHARED((N,), jnp.float32),
            pltpu.VMEM((3, BLK), jnp.int32),
            pltpu.VMEM((3, BLK), jnp.int32),
            pltpu.VMEM((3, BLK), jnp.float32),
            pltpu.VMEM((3, BLK), jnp.float32),
            pltpu.VMEM((2, BLK, D), jnp.float32),
            pltpu.VMEM((2, BLK, D), jnp.float32),
            pltpu.VMEM((2, BLK), jnp.float32),
            pltpu.VMEM((D,), jnp.float32),
            pltpu.VMEM((D,), jnp.float32),
            pltpu.SemaphoreType.DMA,
            pltpu.SemaphoreType.DMA,
            pltpu.SemaphoreType.DMA,
            pltpu.SemaphoreType.DMA,
            pltpu.SemaphoreType.DMA,
            pltpu.SemaphoreType.DMA,
        ],
    )
    return kern(xl, xr, src, dst, ea, vm, w, att)


_RB = 1000
_GRID = N // _RB


def _prep_math(h, Wl, bl, Wr, br, wv, att):
    xl = lax.dot_general(h, Wl, (((1,), (1,)), ((), ())),
                         preferred_element_type=jnp.float32) + bl
    xr = lax.dot_general(h, Wr, (((1,), (1,)), ((), ())),
                         preferred_element_type=jnp.float32) + br
    z = xl + xr + wv
    z = jnp.maximum(z, NEG * z)
    a = jnp.sum(z * att, axis=1, keepdims=True)
    p = jnp.exp(a)
    return xl, xr, xl * p, jnp.broadcast_to(p, (h.shape[0], 16))


def _tc_prep_body(x_ref, wl_ref, bl_ref, wr_ref, br_ref, w_ref, att_ref,
                  xl_o, xr_o, sA_o, sB_o):
    xl, xr, sA, sB = _prep_math(x_ref[...], wl_ref[...], bl_ref[...],
                                wr_ref[...], br_ref[...], w_ref[...],
                                att_ref[...])
    xl_o[...] = xl
    xr_o[...] = xr
    sA_o[...] = sA
    sB_o[...] = sB


def _wspec():
    return pl.BlockSpec((D, D), lambda i: (0, 0))


def _vspec():
    return pl.BlockSpec((1, D), lambda i: (0, 0))


def _rspec(w=D):
    return pl.BlockSpec((_RB, w), lambda i: (i, 0))


def _aspec(w=D):
    return pl.BlockSpec((NC, _RB, w), lambda i: (0, i, 0))


def _bspec():
    return pl.BlockSpec((NC, _RB, 1), lambda i: (0, i, 0))


def _den_slice(aB_ref, sB_ref):
    aB = aB_ref[...]
    return aB[0] + aB[1] + sB_ref[:, 0:1]


def _tc_prep(x, Wl, bl, Wr, br, w, att):
    return pl.pallas_call(
        _tc_prep_body,
        grid=(_GRID,),
        in_specs=[_rspec(), _wspec(), _vspec(), _wspec(), _vspec(),
                  _vspec(), _vspec()],
        out_specs=[_rspec(), _rspec(), _rspec(), _rspec(16)],
        out_shape=[
            jax.ShapeDtypeStruct((N, D), jnp.float32),
            jax.ShapeDtypeStruct((N, D), jnp.float32),
            jax.ShapeDtypeStruct((N, D), jnp.float32),
            jax.ShapeDtypeStruct((N, 16), jnp.float32),
        ],
    )(x, Wl, bl.reshape(1, D), Wr, br.reshape(1, D), w.reshape(1, D),
      att.reshape(1, D))


def _tc_combine_prep_body(aA_ref, aB_ref, sA_ref, sB_ref, bias_ref,
                          wl_ref, bl_ref, wr_ref, br_ref, w_ref, att_ref,
                          xl_o, xr_o, sA_o, sB_o):
    aA = aA_ref[...]
    num = aA[0] + aA[1] + sA_ref[...]
    den = _den_slice(aB_ref, sB_ref)
    h = num / den + bias_ref[...]
    xl, xr, sA, sB = _prep_math(h, wl_ref[...], bl_ref[...], wr_ref[...],
                                br_ref[...], w_ref[...], att_ref[...])
    xl_o[...] = xl
    xr_o[...] = xr
    sA_o[...] = sA
    sB_o[...] = sB


def _tc_combine_prep(aA, aB, sA, sB, bias, Wl, bl, Wr, br, w, att):
    return pl.pallas_call(
        _tc_combine_prep_body,
        grid=(_GRID,),
        in_specs=[_aspec(), _bspec(), _rspec(), _rspec(16), _vspec(),
                  _wspec(), _vspec(), _wspec(), _vspec(), _vspec(), _vspec()],
        out_specs=[_rspec(), _rspec(), _rspec(), _rspec(16)],
        out_shape=[
            jax.ShapeDtypeStruct((N, D), jnp.float32),
            jax.ShapeDtypeStruct((N, D), jnp.float32),
            jax.ShapeDtypeStruct((N, D), jnp.float32),
            jax.ShapeDtypeStruct((N, 16), jnp.float32),
        ],
    )(aA, aB, sA, sB, bias.reshape(1, D), Wl, bl.reshape(1, D), Wr,
      br.reshape(1, D), w.reshape(1, D), att.reshape(1, D))


def _tc_final_body(aA_ref, aB_ref, sA_ref, sB_ref, bias_ref, out_o):
    aA = aA_ref[...]
    num = aA[0] + aA[1] + sA_ref[...]
    den = _den_slice(aB_ref, sB_ref)
    out_o[...] = num / den + bias_ref[...]


def _tc_final(aA, aB, sA, sB, bias):
    return pl.pallas_call(
        _tc_final_body,
        grid=(_GRID,),
        in_specs=[_aspec(), _bspec(), _rspec(), _rspec(16), _vspec()],
        out_specs=_rspec(),
        out_shape=jax.ShapeDtypeStruct((N, D), jnp.float32),
    )(aA, aB, sA, sB, bias.reshape(1, D))


def kernel(x, edge_index, edge_attr, edge_mask_ii, edge_mask_uiu,
           edge_mask_train, node_mask_item, Wl_ii, bl_ii, Wr_ii, br_ii,
           We_ii, att_ii, bias_ii, Wl_uiu, bl_uiu, Wr_uiu, br_uiu, We_uiu,
           att_uiu, bias_uiu):
    src = edge_index[0]
    dst = edge_index[1]
    ea = edge_attr[:, 0]
    vm1 = edge_mask_ii.astype(jnp.float32)
    vm2 = (edge_mask_uiu & edge_mask_train).astype(jnp.float32)
    w1 = We_ii[:, 0]
    w2 = We_uiu[:, 0]

    xl1, xr1, sA1, sB1 = _tc_prep(x, Wl_ii, bl_ii, Wr_ii, br_ii, w1, att_ii)
    aA1, aB1 = _sc_edge_pass(xl1, xr1, src, dst, ea, vm1, w1, att_ii)
    xl2, xr2, sA2, sB2 = _tc_combine_prep(aA1, aB1.reshape(NC, N, 1), sA1,
                                          sB1, bias_ii, Wl_uiu, bl_uiu,
                                          Wr_uiu, br_uiu, w2, att_uiu)
    aA2, aB2 = _sc_edge_pass(xl2, xr2, src, dst, ea, vm2, w2, att_uiu)
    return _tc_final(aA2, aB2.reshape(NC, N, 1), sA2, sB2, bias_uiu)
